# baseline (speedup 1.0000x reference)
import functools

import jax
import jax.numpy as jnp
from jax import lax
from jax.experimental import pallas as pl
from jax.experimental.pallas import tpu as pltpu
from jax.experimental.pallas import tpu_sc as plsc

_NC = 2
_NS = 16
_NW = _NC * _NS


@functools.lru_cache(maxsize=None)
def _make_segsum(n, d, e, chunk):
    ept = e // _NW
    assert ept * _NW == e and ept % chunk == 0
    nit = ept // chunk
    assert nit >= 4
    rpt = (n // _NS) // 8 * 8
    rem = n - rpt * _NS
    assert rem % 8 == 0
    mesh = plsc.VectorSubcoreMesh(core_axis_name="c", subcore_axis_name="s")

    @functools.partial(
        pl.kernel,
        mesh=mesh,
        out_type=jax.ShapeDtypeStruct((_NC, n, d), jnp.float32),
        scratch_types=[
            pltpu.VMEM((ept,), jnp.int32),
            pltpu.VMEM((ept,), jnp.int32),
            pltpu.VMEM((chunk,), jnp.int32),
            pltpu.VMEM((chunk,), jnp.int32),
            pltpu.VMEM((chunk, d), jnp.float32),
            pltpu.VMEM((chunk, d), jnp.float32),
            pltpu.VMEM_SHARED((n, d), jnp.float32),
            pltpu.SemaphoreType.DMA,
            pltpu.SemaphoreType.DMA,
        ],
    )
    def seg(table_hbm, src_hbm, dst_hbm, zeros_hbm, out_hbm,
            src_all, dst_all, dst_c0, dst_c1, rows0, rows1, acc_sh,
            sem0, sem1):
        c = lax.axis_index("c")
        s = lax.axis_index("s")
        wid = s * _NC + c
        zoff = pl.multiple_of(s * rpt, 8)
        pltpu.sync_copy(zeros_hbm.at[pl.ds(zoff, rpt)],
                        acc_sh.at[pl.ds(zoff, rpt)])
        if rem:
            @pl.when(s == 0)
            def _():
                pltpu.sync_copy(zeros_hbm.at[pl.ds(n - rem, rem)],
                                acc_sh.at[pl.ds(n - rem, rem)])
        base = wid * ept
        pltpu.sync_copy(src_hbm.at[pl.ds(base, ept)], src_all)
        pltpu.sync_copy(dst_hbm.at[pl.ds(base, ept)], dst_all)
        plsc.subcore_barrier()

        dst_c = (dst_c0, dst_c1)
        rows = (rows0, rows1)
        sems = (sem0, sem1)

        def start(i, b):
            off = pl.multiple_of(i * chunk, 8)
            pltpu.async_copy(
                table_hbm.at[src_all.at[pl.ds(off, chunk)]], rows[b], sems[b])

        def drain(i, b):
            off = pl.multiple_of(i * chunk, 8)
            for j in range(chunk // 16):
                dst_c[b][pl.ds(j * 16, 16)] = dst_all[pl.ds(off + j * 16, 16)]
            pltpu.make_async_copy(
                table_hbm.at[src_all.at[pl.ds(off, chunk)]], rows[b],
                sems[b]).wait()
            pltpu.sync_copy(rows[b], acc_sh.at[dst_c[b]], add=True)

        start(0, 0)
        npairs = (nit - 2) // 2

        def body(k, carry):
            i = pl.multiple_of(k * 2, 2)
            start(i + 1, 1)
            drain(i, 0)
            start(i + 2, 0)
            drain(i + 1, 1)
            return carry

        lax.fori_loop(0, npairs, body, 0)
        if nit % 2 == 0:
            start(nit - 1, 1)
            drain(nit - 2, 0)
            drain(nit - 1, 1)
        else:
            start(nit - 2, 1)
            drain(nit - 3, 0)
            start(nit - 1, 0)
            drain(nit - 2, 1)
            drain(nit - 1, 0)
        plsc.subcore_barrier()
        pltpu.sync_copy(acc_sh.at[pl.ds(zoff, rpt)],
                        out_hbm.at[c, pl.ds(zoff, rpt)])
        if rem:
            @pl.when(s == 0)
            def _():
                pltpu.sync_copy(acc_sh.at[pl.ds(n - rem, rem)],
                                out_hbm.at[c, pl.ds(n - rem, rem)])

    return seg


@functools.lru_cache(maxsize=None)
def _make_segsum4(n, d, e, chunk, nt):
    ept = e // _NW
    assert ept * _NW == e and ept % chunk == 0
    nit = ept // chunk
    assert nit >= 4
    rpt = (n // _NS) // 8 * 8
    rem = n - rpt * _NS
    assert rem % 8 == 0
    mesh = plsc.VectorSubcoreMesh(core_axis_name="c", subcore_axis_name="s")

    @functools.partial(
        pl.kernel,
        mesh=mesh,
        out_type=jax.ShapeDtypeStruct((nt, _NC, n, d), jnp.float32),
        scratch_types=[
            pltpu.VMEM((ept,), jnp.int32),
            pltpu.VMEM((ept,), jnp.int32),
            pltpu.VMEM((chunk,), jnp.int32),
            pltpu.VMEM((chunk,), jnp.int32),
            pltpu.VMEM((chunk, d), jnp.float32),
            pltpu.VMEM((chunk, d), jnp.float32),
            pltpu.VMEM_SHARED((n, d), jnp.float32),
            pltpu.SemaphoreType.DMA,
            pltpu.SemaphoreType.DMA,
        ],
    )
    def seg4(*refs):
        tables = refs[0:nt]
        (src_hbm, dst_hbm, zeros_hbm, out_hbm,
         src_all, dst_all, dst_c0, dst_c1, rows0, rows1, acc_sh,
         sem0, sem1) = refs[nt:]
        c = lax.axis_index("c")
        s = lax.axis_index("s")
        wid = s * _NC + c
        zoff = pl.multiple_of(s * rpt, 8)
        base = wid * ept
        pltpu.sync_copy(src_hbm.at[pl.ds(base, ept)], src_all)
        pltpu.sync_copy(dst_hbm.at[pl.ds(base, ept)], dst_all)

        dst_c = (dst_c0, dst_c1)
        rows = (rows0, rows1)
        sems = (sem0, sem1)

        for f in range(nt):
            table_hbm = tables[f]
            pltpu.sync_copy(zeros_hbm.at[pl.ds(zoff, rpt)],
                            acc_sh.at[pl.ds(zoff, rpt)])
            if rem:
                @pl.when(s == 0)
                def _():
                    pltpu.sync_copy(zeros_hbm.at[pl.ds(n - rem, rem)],
                                    acc_sh.at[pl.ds(n - rem, rem)])
            plsc.subcore_barrier()

            def start(i, b):
                off = pl.multiple_of(i * chunk, 8)
                pltpu.async_copy(
                    table_hbm.at[src_all.at[pl.ds(off, chunk)]],
                    rows[b], sems[b])

            def drain(i, b):
                off = pl.multiple_of(i * chunk, 8)
                for j in range(chunk // 16):
                    dst_c[b][pl.ds(j * 16, 16)] = (
                        dst_all[pl.ds(off + j * 16, 16)])
                pltpu.make_async_copy(
                    table_hbm.at[src_all.at[pl.ds(off, chunk)]], rows[b],
                    sems[b]).wait()
                pltpu.sync_copy(rows[b], acc_sh.at[dst_c[b]], add=True)

            start(0, 0)
            npairs = (nit - 2) // 2

            def body(k, carry):
                i = pl.multiple_of(k * 2, 2)
                start(i + 1, 1)
                drain(i, 0)
                start(i + 2, 0)
                drain(i + 1, 1)
                return carry

            lax.fori_loop(0, npairs, body, 0)
            if nit % 2 == 0:
                start(nit - 1, 1)
                drain(nit - 2, 0)
                drain(nit - 1, 1)
            else:
                start(nit - 2, 1)
                drain(nit - 3, 0)
                start(nit - 1, 0)
                drain(nit - 2, 1)
                drain(nit - 1, 0)
            plsc.subcore_barrier()
            pltpu.sync_copy(acc_sh.at[pl.ds(zoff, rpt)],
                            out_hbm.at[f, c, pl.ds(zoff, rpt)])
            if rem:
                @pl.when(s == 0)
                def _():
                    pltpu.sync_copy(acc_sh.at[pl.ds(n - rem, rem)],
                                    out_hbm.at[f, c, pl.ds(n - rem, rem)])

    return seg4


@functools.lru_cache(maxsize=None)
def _make_gatherprod(n, dp, et_pad, chunk):
    ept = et_pad // _NW
    assert ept * _NW == et_pad and ept % chunk == 0
    nit = ept // chunk
    assert nit >= 4
    mesh = plsc.VectorSubcoreMesh(core_axis_name="c", subcore_axis_name="s")

    @functools.partial(
        pl.kernel,
        mesh=mesh,
        out_type=jax.ShapeDtypeStruct((et_pad, 2 * dp), jnp.float32),
        scratch_types=[
            pltpu.VMEM((ept,), jnp.int32),
            pltpu.VMEM((ept,), jnp.int32),
            pltpu.VMEM((chunk, dp), jnp.int32),
            pltpu.VMEM((chunk, dp), jnp.int32),
            pltpu.VMEM((chunk, dp), jnp.int32),
            pltpu.VMEM((chunk, dp), jnp.int32),
            pltpu.VMEM((chunk, 2 * dp), jnp.float32),
            pltpu.VMEM((chunk, 2 * dp), jnp.float32),
            pltpu.SemaphoreType.DMA,
            pltpu.SemaphoreType.DMA,
            pltpu.SemaphoreType.DMA,
            pltpu.SemaphoreType.DMA,
        ],
    )
    def g2(h_hbm, a_hbm, b_hbm, out_hbm,
           ia_all, ib_all, ra0, ra1, rb0, rb1, zo0, zo1, sa0, sa1, sb0, sb1):
        c = lax.axis_index("c")
        s = lax.axis_index("s")
        wid = s * _NC + c
        base = wid * ept
        pltpu.sync_copy(a_hbm.at[pl.ds(base, ept)], ia_all)
        pltpu.sync_copy(b_hbm.at[pl.ds(base, ept)], ib_all)

        ra = (ra0, ra1)
        rb = (rb0, rb1)
        zo = (zo0, zo1)
        sa = (sa0, sa1)
        sb = (sb0, sb1)

        def start(i, b):
            off = pl.multiple_of(i * chunk, 8)
            pltpu.async_copy(h_hbm.at[ia_all.at[pl.ds(off, chunk)]],
                             ra[b], sa[b])
            pltpu.async_copy(h_hbm.at[ib_all.at[pl.ds(off, chunk)]],
                             rb[b], sb[b])

        def drain(i, b):
            off = pl.multiple_of(i * chunk, 8)
            hoff = base + off
            pltpu.make_async_copy(
                h_hbm.at[ia_all.at[pl.ds(off, chunk)]], ra[b], sa[b]).wait()
            pltpu.make_async_copy(
                h_hbm.at[ib_all.at[pl.ds(off, chunk)]], rb[b], sb[b]).wait()

            def prod_row(r, carry):
                for cc in range(dp // 16):
                    sl = pl.ds(cc * 16, 16)
                    wa = ra[b][r, sl]
                    wb = rb[b][r, sl]
                    la = lax.bitcast_convert_type(wa << 16, jnp.float32)
                    lb = lax.bitcast_convert_type(wb << 16, jnp.float32)
                    ha = lax.bitcast_convert_type((wa >> 16) << 16,
                                                  jnp.float32)
                    hb = lax.bitcast_convert_type((wb >> 16) << 16,
                                                  jnp.float32)
                    zo[b][r, sl] = la * lb
                    zo[b][r, pl.ds(dp + cc * 16, 16)] = ha * hb
                return carry

            lax.fori_loop(0, chunk, prod_row, 0)
            pltpu.sync_copy(zo[b], out_hbm.at[pl.ds(hoff, chunk)])

        start(0, 0)
        npairs = (nit - 2) // 2

        def body(k, carry):
            i = pl.multiple_of(k * 2, 2)
            start(i + 1, 1)
            drain(i, 0)
            start(i + 2, 0)
            drain(i + 1, 1)
            return carry

        lax.fori_loop(0, npairs, body, 0)
        if nit % 2 == 0:
            start(nit - 1, 1)
            drain(nit - 2, 0)
            drain(nit - 1, 1)
        else:
            start(nit - 2, 1)
            drain(nit - 3, 0)
            start(nit - 1, 0)
            drain(nit - 2, 1)
            drain(nit - 1, 0)

    return g2


def _relu(v):
    return jnp.maximum(v, 0.0)


@functools.lru_cache(maxsize=None)
def _make_mlp1(n, din, hid, blk):
    nb = n // blk

    def body(x_ref, agg_ref, w1a_ref, b1a_ref, w1b_ref, b1b_ref, sc_ref,
             t_ref, sums_ref, acc_ref):
        i = pl.program_id(0)
        m = sc_ref[...] * x_ref[...] + agg_ref[0] + agg_ref[1]
        t = _relu(jnp.dot(m, w1a_ref[...], preferred_element_type=jnp.float32)
                  + b1a_ref[...])
        t = _relu(jnp.dot(t, w1b_ref[...], preferred_element_type=jnp.float32)
                  + b1b_ref[...])
        t_ref[...] = t

        @pl.when(i == 0)
        def _():
            acc_ref[...] = jnp.zeros_like(acc_ref)

        acc_ref[0:1, :] += jnp.sum(t, axis=0, keepdims=True)
        acc_ref[1:2, :] += jnp.sum(t * t, axis=0, keepdims=True)

        @pl.when(i == nb - 1)
        def _():
            sums_ref[...] = acc_ref[...]

    return pl.pallas_call(
        body,
        grid=(nb,),
        in_specs=[
            pl.BlockSpec((blk, din), lambda i: (i, 0)),
            pl.BlockSpec((2, blk, din), lambda i: (0, i, 0)),
            pl.BlockSpec((din, hid), lambda i: (0, 0)),
            pl.BlockSpec((1, hid), lambda i: (0, 0)),
            pl.BlockSpec((hid, hid), lambda i: (0, 0)),
            pl.BlockSpec((1, hid), lambda i: (0, 0)),
            pl.BlockSpec((1, 1), lambda i: (0, 0)),
        ],
        out_specs=[
            pl.BlockSpec((blk, hid), lambda i: (i, 0)),
            pl.BlockSpec((2, hid), lambda i: (0, 0)),
        ],
        out_shape=[
            jax.ShapeDtypeStruct((n, hid), jnp.float32),
            jax.ShapeDtypeStruct((2, hid), jnp.float32),
        ],
        scratch_shapes=[pltpu.VMEM((2, hid), jnp.float32)],
    )


@functools.lru_cache(maxsize=None)
def _make_bn_split(n, hid, blk):
    nb = n // blk
    nf = hid // 128
    inv_n = 1.0 / n

    def body(t_ref, sums_ref, g_ref, be_ref, *out_refs):
        mean = sums_ref[0:1, :] * inv_n
        var = sums_ref[1:2, :] * inv_n - mean * mean
        hn = (t_ref[...] - mean) * lax.rsqrt(var + 1e-5) * g_ref[...] + be_ref[...]
        for f in range(nf):
            out_refs[f][...] = hn[:, f * 128:(f + 1) * 128]

    return pl.pallas_call(
        body,
        grid=(nb,),
        in_specs=[
            pl.BlockSpec((blk, hid), lambda i: (i, 0)),
            pl.BlockSpec((2, hid), lambda i: (0, 0)),
            pl.BlockSpec((1, hid), lambda i: (0, 0)),
            pl.BlockSpec((1, hid), lambda i: (0, 0)),
        ],
        out_specs=[pl.BlockSpec((blk, 128), lambda i: (i, 0))] * nf,
        out_shape=[jax.ShapeDtypeStruct((n, 128), jnp.float32)] * nf,
    )


@functools.lru_cache(maxsize=None)
def _make_mlp2(n, hid, blk):
    nb = n // blk
    nf = hid // 128

    def body(*refs):
        h_refs = refs[0:nf]
        agg_ref, w2_ref, b2_ref, sc_ref, t_ref, sums_ref, acc_ref = refs[nf:]
        i = pl.program_id(0)
        t = b2_ref[...]
        for f in range(nf):
            m = sc_ref[...] * h_refs[f][...] + agg_ref[f, 0] + agg_ref[f, 1]
            t = t + jnp.dot(m, w2_ref[f], preferred_element_type=jnp.float32)
        t = _relu(t)
        t_ref[...] = t

        @pl.when(i == 0)
        def _():
            acc_ref[...] = jnp.zeros_like(acc_ref)

        acc_ref[0:1, :] += jnp.sum(t, axis=0, keepdims=True)
        acc_ref[1:2, :] += jnp.sum(t * t, axis=0, keepdims=True)

        @pl.when(i == nb - 1)
        def _():
            sums_ref[...] = acc_ref[...]

    return pl.pallas_call(
        body,
        grid=(nb,),
        in_specs=(
            [pl.BlockSpec((blk, 128), lambda i: (i, 0))] * nf
            + [
                pl.BlockSpec((nf, 2, blk, 128), lambda i: (0, 0, i, 0)),
                pl.BlockSpec((nf, 128, hid), lambda i: (0, 0, 0)),
                pl.BlockSpec((1, hid), lambda i: (0, 0)),
                pl.BlockSpec((1, 1), lambda i: (0, 0)),
            ]
        ),
        out_specs=[
            pl.BlockSpec((blk, hid), lambda i: (i, 0)),
            pl.BlockSpec((2, hid), lambda i: (0, 0)),
        ],
        out_shape=[
            jax.ShapeDtypeStruct((n, hid), jnp.float32),
            jax.ShapeDtypeStruct((2, hid), jnp.float32),
        ],
        scratch_shapes=[pltpu.VMEM((2, hid), jnp.float32)],
    )


@functools.lru_cache(maxsize=None)
def _make_bn_lin(n, hid, blk):
    nb = n // blk
    inv_n = 1.0 / n

    def body(t_ref, sums_ref, g_ref, be_ref, wl_ref, bl_ref, h3_ref):
        mean = sums_ref[0:1, :] * inv_n
        var = sums_ref[1:2, :] * inv_n - mean * mean
        hn = (t_ref[...] - mean) * lax.rsqrt(var + 1e-5) * g_ref[...] + be_ref[...]
        h3 = _relu(
            jnp.dot(hn, wl_ref[...], preferred_element_type=jnp.float32)
            + bl_ref[...])
        lob = lax.bitcast_convert_type(h3[:, :hid // 2], jnp.int32)
        hib = lax.bitcast_convert_type(h3[:, hid // 2:], jnp.int32)
        h3_ref[...] = ((lob + 0x8000) >> 16) | (
            ((hib + 0x8000) >> 16) << 16)

    return pl.pallas_call(
        body,
        grid=(nb,),
        in_specs=[
            pl.BlockSpec((blk, hid), lambda i: (i, 0)),
            pl.BlockSpec((2, hid), lambda i: (0, 0)),
            pl.BlockSpec((1, hid), lambda i: (0, 0)),
            pl.BlockSpec((1, hid), lambda i: (0, 0)),
            pl.BlockSpec((hid, hid), lambda i: (0, 0)),
            pl.BlockSpec((1, hid), lambda i: (0, 0)),
        ],
        out_specs=pl.BlockSpec((blk, hid // 2), lambda i: (i, 0)),
        out_shape=jax.ShapeDtypeStruct((n, hid // 2), jnp.int32),
    )


@functools.lru_cache(maxsize=None)
def _make_score(et, et_pad, hid, out_dim, blk):
    nb = et // blk

    def body(z_ref, wf_ref, bf_ref, o_ref):
        o_ref[...] = jnp.dot(z_ref[...], wf_ref[...],
                             preferred_element_type=jnp.float32) + bf_ref[...]

    return pl.pallas_call(
        body,
        grid=(nb,),
        in_specs=[
            pl.BlockSpec((blk, hid), lambda i: (i, 0)),
            pl.BlockSpec((hid, out_dim), lambda i: (0, 0)),
            pl.BlockSpec((1, out_dim), lambda i: (0, 0)),
        ],
        out_specs=pl.BlockSpec((blk, out_dim), lambda i: (i, 0)),
        out_shape=jax.ShapeDtypeStruct((et, out_dim), jnp.float32),
    )


def kernel(x, edge_index, train_edge_index, W1a, b1a, W1b, b1b, g1, be1, eps1,
           W2, b2, g2, be2, eps2, Wl, bl, Wf, bf):
    n, din = x.shape
    e = edge_index.shape[1]
    et = train_edge_index.shape[1]
    hid = W1a.shape[1]
    out_dim = Wf.shape[1]
    nf = hid // 128
    blk = 1000

    src = edge_index[0]
    dst = edge_index[1]
    zeros_nd = jnp.zeros((n, 128), jnp.float32)

    scale1 = (1.0 + eps1).reshape(1, 1)
    scale2 = (1.0 + eps2).reshape(1, 1)
    b1a2, b1b2, g12, be12 = (v.reshape(1, hid) for v in (b1a, b1b, g1, be1))
    b22, g22, be22, bl2 = (v.reshape(1, hid) for v in (b2, g2, be2, bl))
    bf2 = bf.reshape(1, out_dim)
    w2r = W2.reshape(nf, 128, hid)

    segsum = _make_segsum(n, 128, e, 80)

    agg1 = segsum(x, src, dst, zeros_nd)
    t1, sums1 = _make_mlp1(n, din, hid, blk)(x, agg1, W1a, b1a2, W1b, b1b2,
                                             scale1)
    hfeats = _make_bn_split(n, hid, blk)(t1, sums1, g12, be12)

    agg2 = _make_segsum4(n, 128, e, 80, nf)(*hfeats, src, dst, zeros_nd)
    t2, sums2 = _make_mlp2(n, hid, blk)(*hfeats, agg2, w2r, b22, scale2)
    h3 = _make_bn_lin(n, hid, blk)(t2, sums2, g22, be22, Wl, bl2)

    ept_chunk = 40 * _NW
    et_pad = ((et + ept_chunk - 1) // ept_chunk) * ept_chunk
    tei = jnp.pad(train_edge_index, ((0, 0), (0, et_pad - et)))
    z = _make_gatherprod(n, hid // 2, et_pad, 40)(h3, tei[0], tei[1])
    out = _make_score(et, et_pad, hid, out_dim, blk)(z, Wf, bf2)
    return out


# speedup vs baseline: 1.0544x; 1.0544x over previous
import functools

import jax
import jax.numpy as jnp
from jax import lax
from jax.experimental import pallas as pl
from jax.experimental.pallas import tpu as pltpu
from jax.experimental.pallas import tpu_sc as plsc

_NC = 2
_NS = 16
_NW = _NC * _NS


@functools.lru_cache(maxsize=None)
def _make_segsum(n, d, e, chunk):
    ept = e // _NW
    assert ept * _NW == e and ept % chunk == 0
    nit = ept // chunk
    assert nit >= 4
    rpt = (n // _NS) // 8 * 8
    rem = n - rpt * _NS
    assert rem % 8 == 0
    mesh = plsc.VectorSubcoreMesh(core_axis_name="c", subcore_axis_name="s")

    @functools.partial(
        pl.kernel,
        mesh=mesh,
        out_type=jax.ShapeDtypeStruct((_NC, n, d), jnp.float32),
        scratch_types=[
            pltpu.VMEM((ept,), jnp.int32),
            pltpu.VMEM((ept,), jnp.int32),
            pltpu.VMEM((chunk,), jnp.int32),
            pltpu.VMEM((chunk,), jnp.int32),
            pltpu.VMEM((chunk, d), jnp.float32),
            pltpu.VMEM((chunk, d), jnp.float32),
            pltpu.VMEM_SHARED((n, d), jnp.float32),
            pltpu.SemaphoreType.DMA,
            pltpu.SemaphoreType.DMA,
        ],
    )
    def seg(table_hbm, src_hbm, dst_hbm, zeros_hbm, out_hbm,
            src_all, dst_all, dst_c0, dst_c1, rows0, rows1, acc_sh,
            sem0, sem1):
        c = lax.axis_index("c")
        s = lax.axis_index("s")
        wid = s * _NC + c
        zoff = pl.multiple_of(s * rpt, 8)
        pltpu.sync_copy(zeros_hbm.at[pl.ds(zoff, rpt)],
                        acc_sh.at[pl.ds(zoff, rpt)])
        if rem:
            @pl.when(s == 0)
            def _():
                pltpu.sync_copy(zeros_hbm.at[pl.ds(n - rem, rem)],
                                acc_sh.at[pl.ds(n - rem, rem)])
        base = wid * ept
        pltpu.sync_copy(src_hbm.at[pl.ds(base, ept)], src_all)
        pltpu.sync_copy(dst_hbm.at[pl.ds(base, ept)], dst_all)
        plsc.subcore_barrier()

        dst_c = (dst_c0, dst_c1)
        rows = (rows0, rows1)
        sems = (sem0, sem1)

        def start(i, b):
            off = pl.multiple_of(i * chunk, 8)
            pltpu.async_copy(
                table_hbm.at[src_all.at[pl.ds(off, chunk)]], rows[b], sems[b])

        def drain(i, b):
            off = pl.multiple_of(i * chunk, 8)
            for j in range(chunk // 16):
                dst_c[b][pl.ds(j * 16, 16)] = dst_all[pl.ds(off + j * 16, 16)]
            pltpu.make_async_copy(
                table_hbm.at[src_all.at[pl.ds(off, chunk)]], rows[b],
                sems[b]).wait()
            pltpu.sync_copy(rows[b], acc_sh.at[dst_c[b]], add=True)

        start(0, 0)
        npairs = (nit - 2) // 2

        def body(k, carry):
            i = pl.multiple_of(k * 2, 2)
            start(i + 1, 1)
            drain(i, 0)
            start(i + 2, 0)
            drain(i + 1, 1)
            return carry

        lax.fori_loop(0, npairs, body, 0)
        if nit % 2 == 0:
            start(nit - 1, 1)
            drain(nit - 2, 0)
            drain(nit - 1, 1)
        else:
            start(nit - 2, 1)
            drain(nit - 3, 0)
            start(nit - 1, 0)
            drain(nit - 2, 1)
            drain(nit - 1, 0)
        plsc.subcore_barrier()
        pltpu.sync_copy(acc_sh.at[pl.ds(zoff, rpt)],
                        out_hbm.at[c, pl.ds(zoff, rpt)])
        if rem:
            @pl.when(s == 0)
            def _():
                pltpu.sync_copy(acc_sh.at[pl.ds(n - rem, rem)],
                                out_hbm.at[c, pl.ds(n - rem, rem)])

    return seg


@functools.lru_cache(maxsize=None)
def _make_segsum4(n, d, e, chunk, nt):
    ept = e // _NW
    assert ept * _NW == e and ept % chunk == 0
    nit = ept // chunk
    assert nit >= 4
    rpt = (n // _NS) // 8 * 8
    rem = n - rpt * _NS
    assert rem % 8 == 0
    mesh = plsc.VectorSubcoreMesh(core_axis_name="c", subcore_axis_name="s")

    @functools.partial(
        pl.kernel,
        mesh=mesh,
        out_type=jax.ShapeDtypeStruct((nt, _NC, n, d), jnp.float32),
        scratch_types=[
            pltpu.VMEM((ept,), jnp.int32),
            pltpu.VMEM((ept,), jnp.int32),
            pltpu.VMEM((chunk,), jnp.int32),
            pltpu.VMEM((chunk,), jnp.int32),
            pltpu.VMEM((chunk, d), jnp.float32),
            pltpu.VMEM((chunk, d), jnp.float32),
            pltpu.VMEM_SHARED((n, d), jnp.float32),
            pltpu.SemaphoreType.DMA,
            pltpu.SemaphoreType.DMA,
        ],
    )
    def seg4(*refs):
        tables = refs[0:nt]
        (src_hbm, dst_hbm, zeros_hbm, out_hbm,
         src_all, dst_all, dst_c0, dst_c1, rows0, rows1, acc_sh,
         sem0, sem1) = refs[nt:]
        c = lax.axis_index("c")
        s = lax.axis_index("s")
        wid = s * _NC + c
        zoff = pl.multiple_of(s * rpt, 8)
        base = wid * ept
        pltpu.sync_copy(src_hbm.at[pl.ds(base, ept)], src_all)
        pltpu.sync_copy(dst_hbm.at[pl.ds(base, ept)], dst_all)

        dst_c = (dst_c0, dst_c1)
        rows = (rows0, rows1)
        sems = (sem0, sem1)

        for f in range(nt):
            table_hbm = tables[f]
            pltpu.sync_copy(zeros_hbm.at[pl.ds(zoff, rpt)],
                            acc_sh.at[pl.ds(zoff, rpt)])
            if rem:
                @pl.when(s == 0)
                def _():
                    pltpu.sync_copy(zeros_hbm.at[pl.ds(n - rem, rem)],
                                    acc_sh.at[pl.ds(n - rem, rem)])
            plsc.subcore_barrier()

            def start(i, b):
                off = pl.multiple_of(i * chunk, 8)
                pltpu.async_copy(
                    table_hbm.at[src_all.at[pl.ds(off, chunk)]],
                    rows[b], sems[b])

            def drain(i, b):
                off = pl.multiple_of(i * chunk, 8)
                for j in range(chunk // 16):
                    dst_c[b][pl.ds(j * 16, 16)] = (
                        dst_all[pl.ds(off + j * 16, 16)])
                pltpu.make_async_copy(
                    table_hbm.at[src_all.at[pl.ds(off, chunk)]], rows[b],
                    sems[b]).wait()
                pltpu.sync_copy(rows[b], acc_sh.at[dst_c[b]], add=True)

            start(0, 0)
            npairs = (nit - 2) // 2

            def body(k, carry):
                i = pl.multiple_of(k * 2, 2)
                start(i + 1, 1)
                drain(i, 0)
                start(i + 2, 0)
                drain(i + 1, 1)
                return carry

            lax.fori_loop(0, npairs, body, 0)
            if nit % 2 == 0:
                start(nit - 1, 1)
                drain(nit - 2, 0)
                drain(nit - 1, 1)
            else:
                start(nit - 2, 1)
                drain(nit - 3, 0)
                start(nit - 1, 0)
                drain(nit - 2, 1)
                drain(nit - 1, 0)
            plsc.subcore_barrier()
            pltpu.sync_copy(acc_sh.at[pl.ds(zoff, rpt)],
                            out_hbm.at[f, c, pl.ds(zoff, rpt)])
            if rem:
                @pl.when(s == 0)
                def _():
                    pltpu.sync_copy(acc_sh.at[pl.ds(n - rem, rem)],
                                    out_hbm.at[f, c, pl.ds(n - rem, rem)])

    return seg4


@functools.lru_cache(maxsize=None)
def _make_gatherprod(n, d, et_pad, chunk, dtype):
    ept = et_pad // _NW
    assert ept * _NW == et_pad and ept % chunk == 0
    nit = ept // chunk
    assert nit >= 4
    mesh = plsc.VectorSubcoreMesh(core_axis_name="c", subcore_axis_name="s")

    @functools.partial(
        pl.kernel,
        mesh=mesh,
        out_type=jax.ShapeDtypeStruct((et_pad, d), dtype),
        scratch_types=[
            pltpu.VMEM((ept,), jnp.int32),
            pltpu.VMEM((ept,), jnp.int32),
            pltpu.VMEM((chunk, d), dtype),
            pltpu.VMEM((chunk, d), dtype),
            pltpu.VMEM((chunk, d), dtype),
            pltpu.VMEM((chunk, d), dtype),
            pltpu.SemaphoreType.DMA,
            pltpu.SemaphoreType.DMA,
            pltpu.SemaphoreType.DMA,
            pltpu.SemaphoreType.DMA,
        ],
    )
    def g2(h_hbm, a_hbm, b_hbm, out_hbm,
           ia_all, ib_all, ra0, ra1, rb0, rb1, sa0, sa1, sb0, sb1):
        c = lax.axis_index("c")
        s = lax.axis_index("s")
        wid = s * _NC + c
        base = wid * ept
        pltpu.sync_copy(a_hbm.at[pl.ds(base, ept)], ia_all)
        pltpu.sync_copy(b_hbm.at[pl.ds(base, ept)], ib_all)

        ra = (ra0, ra1)
        rb = (rb0, rb1)
        sa = (sa0, sa1)
        sb = (sb0, sb1)

        def start(i, b):
            off = pl.multiple_of(i * chunk, 8)
            pltpu.async_copy(h_hbm.at[ia_all.at[pl.ds(off, chunk)]],
                             ra[b], sa[b])
            pltpu.async_copy(h_hbm.at[ib_all.at[pl.ds(off, chunk)]],
                             rb[b], sb[b])

        def drain(i, b):
            off = pl.multiple_of(i * chunk, 8)
            hoff = base + off
            pltpu.make_async_copy(
                h_hbm.at[ia_all.at[pl.ds(off, chunk)]], ra[b], sa[b]).wait()
            pltpu.make_async_copy(
                h_hbm.at[ib_all.at[pl.ds(off, chunk)]], rb[b], sb[b]).wait()

            def prod_row(r, carry):
                for cc in range(d // 16):
                    sl = pl.ds(cc * 16, 16)
                    ra[b][r, sl] = ra[b][r, sl] * rb[b][r, sl]
                return carry

            lax.fori_loop(0, chunk, prod_row, 0)
            pltpu.sync_copy(ra[b], out_hbm.at[pl.ds(hoff, chunk)])

        start(0, 0)
        npairs = (nit - 2) // 2

        def body(k, carry):
            i = pl.multiple_of(k * 2, 2)
            start(i + 1, 1)
            drain(i, 0)
            start(i + 2, 0)
            drain(i + 1, 1)
            return carry

        lax.fori_loop(0, npairs, body, 0)
        if nit % 2 == 0:
            start(nit - 1, 1)
            drain(nit - 2, 0)
            drain(nit - 1, 1)
        else:
            start(nit - 2, 1)
            drain(nit - 3, 0)
            start(nit - 1, 0)
            drain(nit - 2, 1)
            drain(nit - 1, 0)

    return g2


def _relu(v):
    return jnp.maximum(v, 0.0)


@functools.lru_cache(maxsize=None)
def _make_mlp1(n, din, hid, blk):
    nb = n // blk

    def body(x_ref, agg_ref, w1a_ref, b1a_ref, w1b_ref, b1b_ref, sc_ref,
             t_ref, sums_ref, acc_ref):
        i = pl.program_id(0)
        m = sc_ref[...] * x_ref[...] + agg_ref[0] + agg_ref[1]
        t = _relu(jnp.dot(m, w1a_ref[...], preferred_element_type=jnp.float32)
                  + b1a_ref[...])
        t = _relu(jnp.dot(t, w1b_ref[...], preferred_element_type=jnp.float32)
                  + b1b_ref[...])
        t_ref[...] = t

        @pl.when(i == 0)
        def _():
            acc_ref[...] = jnp.zeros_like(acc_ref)

        acc_ref[0:1, :] += jnp.sum(t, axis=0, keepdims=True)
        acc_ref[1:2, :] += jnp.sum(t * t, axis=0, keepdims=True)

        @pl.when(i == nb - 1)
        def _():
            sums_ref[...] = acc_ref[...]

    return pl.pallas_call(
        body,
        grid=(nb,),
        in_specs=[
            pl.BlockSpec((blk, din), lambda i: (i, 0)),
            pl.BlockSpec((2, blk, din), lambda i: (0, i, 0)),
            pl.BlockSpec((din, hid), lambda i: (0, 0)),
            pl.BlockSpec((1, hid), lambda i: (0, 0)),
            pl.BlockSpec((hid, hid), lambda i: (0, 0)),
            pl.BlockSpec((1, hid), lambda i: (0, 0)),
            pl.BlockSpec((1, 1), lambda i: (0, 0)),
        ],
        out_specs=[
            pl.BlockSpec((blk, hid), lambda i: (i, 0)),
            pl.BlockSpec((2, hid), lambda i: (0, 0)),
        ],
        out_shape=[
            jax.ShapeDtypeStruct((n, hid), jnp.float32),
            jax.ShapeDtypeStruct((2, hid), jnp.float32),
        ],
        scratch_shapes=[pltpu.VMEM((2, hid), jnp.float32)],
    )


@functools.lru_cache(maxsize=None)
def _make_bn_split(n, hid, blk):
    nb = n // blk
    nf = hid // 128
    inv_n = 1.0 / n

    def body(t_ref, sums_ref, g_ref, be_ref, *out_refs):
        mean = sums_ref[0:1, :] * inv_n
        var = sums_ref[1:2, :] * inv_n - mean * mean
        hn = (t_ref[...] - mean) * lax.rsqrt(var + 1e-5) * g_ref[...] + be_ref[...]
        for f in range(nf):
            out_refs[f][...] = hn[:, f * 128:(f + 1) * 128]

    return pl.pallas_call(
        body,
        grid=(nb,),
        in_specs=[
            pl.BlockSpec((blk, hid), lambda i: (i, 0)),
            pl.BlockSpec((2, hid), lambda i: (0, 0)),
            pl.BlockSpec((1, hid), lambda i: (0, 0)),
            pl.BlockSpec((1, hid), lambda i: (0, 0)),
        ],
        out_specs=[pl.BlockSpec((blk, 128), lambda i: (i, 0))] * nf,
        out_shape=[jax.ShapeDtypeStruct((n, 128), jnp.float32)] * nf,
    )


@functools.lru_cache(maxsize=None)
def _make_mlp2(n, hid, blk):
    nb = n // blk
    nf = hid // 128

    def body(*refs):
        h_refs = refs[0:nf]
        agg_ref, w2_ref, b2_ref, sc_ref, t_ref, sums_ref, acc_ref = refs[nf:]
        i = pl.program_id(0)
        t = b2_ref[...]
        for f in range(nf):
            m = sc_ref[...] * h_refs[f][...] + agg_ref[f, 0] + agg_ref[f, 1]
            t = t + jnp.dot(m, w2_ref[f], preferred_element_type=jnp.float32)
        t = _relu(t)
        t_ref[...] = t

        @pl.when(i == 0)
        def _():
            acc_ref[...] = jnp.zeros_like(acc_ref)

        acc_ref[0:1, :] += jnp.sum(t, axis=0, keepdims=True)
        acc_ref[1:2, :] += jnp.sum(t * t, axis=0, keepdims=True)

        @pl.when(i == nb - 1)
        def _():
            sums_ref[...] = acc_ref[...]

    return pl.pallas_call(
        body,
        grid=(nb,),
        in_specs=(
            [pl.BlockSpec((blk, 128), lambda i: (i, 0))] * nf
            + [
                pl.BlockSpec((nf, 2, blk, 128), lambda i: (0, 0, i, 0)),
                pl.BlockSpec((nf, 128, hid), lambda i: (0, 0, 0)),
                pl.BlockSpec((1, hid), lambda i: (0, 0)),
                pl.BlockSpec((1, 1), lambda i: (0, 0)),
            ]
        ),
        out_specs=[
            pl.BlockSpec((blk, hid), lambda i: (i, 0)),
            pl.BlockSpec((2, hid), lambda i: (0, 0)),
        ],
        out_shape=[
            jax.ShapeDtypeStruct((n, hid), jnp.float32),
            jax.ShapeDtypeStruct((2, hid), jnp.float32),
        ],
        scratch_shapes=[pltpu.VMEM((2, hid), jnp.float32)],
    )


@functools.lru_cache(maxsize=None)
def _make_bn_lin(n, hid, blk):
    nb = n // blk
    inv_n = 1.0 / n

    def body(t_ref, sums_ref, g_ref, be_ref, wl_ref, bl_ref, h3_ref):
        mean = sums_ref[0:1, :] * inv_n
        var = sums_ref[1:2, :] * inv_n - mean * mean
        hn = (t_ref[...] - mean) * lax.rsqrt(var + 1e-5) * g_ref[...] + be_ref[...]
        h3_ref[...] = _relu(
            jnp.dot(hn, wl_ref[...], preferred_element_type=jnp.float32)
            + bl_ref[...])

    return pl.pallas_call(
        body,
        grid=(nb,),
        in_specs=[
            pl.BlockSpec((blk, hid), lambda i: (i, 0)),
            pl.BlockSpec((2, hid), lambda i: (0, 0)),
            pl.BlockSpec((1, hid), lambda i: (0, 0)),
            pl.BlockSpec((1, hid), lambda i: (0, 0)),
            pl.BlockSpec((hid, hid), lambda i: (0, 0)),
            pl.BlockSpec((1, hid), lambda i: (0, 0)),
        ],
        out_specs=pl.BlockSpec((blk, hid), lambda i: (i, 0)),
        out_shape=jax.ShapeDtypeStruct((n, hid), jnp.float32),
    )


@functools.lru_cache(maxsize=None)
def _make_score(et, et_pad, hid, out_dim, blk):
    nb = et // blk

    def body(z_ref, wf_ref, bf_ref, o_ref):
        o_ref[...] = jnp.dot(z_ref[...], wf_ref[...],
                             preferred_element_type=jnp.float32) + bf_ref[...]

    return pl.pallas_call(
        body,
        grid=(nb,),
        in_specs=[
            pl.BlockSpec((blk, hid), lambda i: (i, 0)),
            pl.BlockSpec((hid, out_dim), lambda i: (0, 0)),
            pl.BlockSpec((1, out_dim), lambda i: (0, 0)),
        ],
        out_specs=pl.BlockSpec((blk, out_dim), lambda i: (i, 0)),
        out_shape=jax.ShapeDtypeStruct((et, out_dim), jnp.float32),
    )


def kernel(x, edge_index, train_edge_index, W1a, b1a, W1b, b1b, g1, be1, eps1,
           W2, b2, g2, be2, eps2, Wl, bl, Wf, bf):
    n, din = x.shape
    e = edge_index.shape[1]
    et = train_edge_index.shape[1]
    hid = W1a.shape[1]
    out_dim = Wf.shape[1]
    nf = hid // 128
    blk = 1000

    src = edge_index[0]
    dst = edge_index[1]
    zeros_nd = jnp.zeros((n, 128), jnp.float32)

    scale1 = (1.0 + eps1).reshape(1, 1)
    scale2 = (1.0 + eps2).reshape(1, 1)
    b1a2, b1b2, g12, be12 = (v.reshape(1, hid) for v in (b1a, b1b, g1, be1))
    b22, g22, be22, bl2 = (v.reshape(1, hid) for v in (b2, g2, be2, bl))
    bf2 = bf.reshape(1, out_dim)
    w2r = W2.reshape(nf, 128, hid)

    segsum = _make_segsum(n, 128, e, 80)

    agg1 = segsum(x, src, dst, zeros_nd)
    t1, sums1 = _make_mlp1(n, din, hid, blk)(x, agg1, W1a, b1a2, W1b, b1b2,
                                             scale1)
    hfeats = _make_bn_split(n, hid, blk)(t1, sums1, g12, be12)

    agg2 = _make_segsum4(n, 128, e, 80, nf)(*hfeats, src, dst, zeros_nd)
    t2, sums2 = _make_mlp2(n, hid, blk)(*hfeats, agg2, w2r, b22, scale2)
    h3 = _make_bn_lin(n, hid, blk)(t2, sums2, g22, be22, Wl, bl2)

    ept_chunk = 40 * _NW
    et_pad = ((et + ept_chunk - 1) // ept_chunk) * ept_chunk
    tei = jnp.pad(train_edge_index, ((0, 0), (0, et_pad - et)))
    z = _make_gatherprod(n, hid, et_pad, 40, jnp.float32)(h3, tei[0], tei[1])
    out = _make_score(et, et_pad, hid, out_dim, blk)(z, Wf, bf2)
    return out
